# Initial kernel scaffold; baseline (speedup 1.0000x reference)
import jax
import jax.numpy as jnp
from jax.experimental import pallas as pl


def kernel(x, table):
    raise NotImplementedError("write your pallas kernel here")


# speedup vs baseline: 1.4766x; 1.4766x over previous
import functools

import jax
import jax.numpy as jnp
from jax import lax
from jax.experimental import pallas as pl
from jax.experimental.pallas import tpu as pltpu
from jax.experimental.pallas import tpu_sc as plsc

VOCAB_DIM = 32
NC, NS = 2, 16
NW = NC * NS
CHUNK = 1600


@functools.partial(jax.jit, static_argnames=("n_rows",))
def _gather_rows(flat_idx, table, n_rows):
    b_per_w = n_rows // NW
    n_chunks = b_per_w // CHUNK
    mesh = plsc.VectorSubcoreMesh(
        core_axis_name="c", subcore_axis_name="s", num_cores=NC, num_subcores=NS
    )

    @functools.partial(
        pl.kernel,
        out_type=jax.ShapeDtypeStruct((n_rows, VOCAB_DIM), jnp.float32),
        mesh=mesh,
        scratch_types=[
            pltpu.VMEM((CHUNK,), jnp.int32),
            pltpu.VMEM((CHUNK, VOCAB_DIM), jnp.float32),
            pltpu.SemaphoreType.DMA,
        ],
        compiler_params=pltpu.CompilerParams(use_tc_tiling_on_sc=False),
    )
    def body(idx_hbm, table_hbm, out_hbm, idx_v, rows_v, sem):
        wid = lax.axis_index("s") * NC + lax.axis_index("c")
        base = wid * b_per_w

        def step(i, _):
            off = base + i * CHUNK
            pltpu.sync_copy(idx_hbm.at[pl.ds(off, CHUNK)], idx_v)
            pltpu.async_copy(table_hbm.at[idx_v], rows_v, sem).wait()
            pltpu.sync_copy(rows_v, out_hbm.at[pl.ds(off, CHUNK)])
            return 0

        lax.fori_loop(0, n_chunks, step, 0)

    return body(flat_idx, table)


def kernel(x, table):
    b, t = x.shape
    flat_idx = x.reshape(-1).astype(jnp.int32)
    out = _gather_rows(flat_idx, table, b * t)
    return out.reshape(b, t, VOCAB_DIM)


# speedup vs baseline: 1.4907x; 1.0096x over previous
import functools

import jax
import jax.numpy as jnp
from jax import lax
from jax.experimental import pallas as pl
from jax.experimental.pallas import tpu as pltpu
from jax.experimental.pallas import tpu_sc as plsc

VOCAB_DIM = 32
NC, NS = 2, 16
NW = NC * NS
CHUNK = 1600


@functools.partial(jax.jit, static_argnames=("n_rows",))
def _gather_rows(flat_idx, table, n_rows):
    b_per_w = n_rows // NW
    n_chunks = b_per_w // CHUNK
    mesh = plsc.VectorSubcoreMesh(
        core_axis_name="c", subcore_axis_name="s", num_cores=NC, num_subcores=NS
    )

    @functools.partial(
        pl.kernel,
        out_type=jax.ShapeDtypeStruct((n_rows, VOCAB_DIM), jnp.float32),
        mesh=mesh,
        scratch_types=[
            pltpu.VMEM((2, CHUNK), jnp.int32),
            pltpu.VMEM((2, CHUNK, VOCAB_DIM), jnp.float32),
            pltpu.SemaphoreType.DMA,
            pltpu.SemaphoreType.DMA,
            pltpu.SemaphoreType.DMA,
            pltpu.SemaphoreType.DMA,
        ],
        compiler_params=pltpu.CompilerParams(use_tc_tiling_on_sc=False),
    )
    def body(idx_hbm, table_hbm, out_hbm, idx_v, rows_v, g0, g1, w0, w1):
        gsem = [g0, g1]
        wsem = [w0, w1]
        wid = lax.axis_index("s") * NC + lax.axis_index("c")
        base = wid * b_per_w

        pltpu.sync_copy(idx_hbm.at[pl.ds(base, CHUNK)], idx_v.at[0])
        gathers = [
            pltpu.async_copy(table_hbm.at[idx_v.at[0]], rows_v.at[0], gsem[0])
        ]
        writes = []
        for i in range(n_chunks):
            bi = i % 2
            if i + 1 < n_chunks:
                nbi = (i + 1) % 2
                if i >= 1:
                    writes[i - 1].wait()
                off = base + (i + 1) * CHUNK
                pltpu.sync_copy(idx_hbm.at[pl.ds(off, CHUNK)], idx_v.at[nbi])
                gathers.append(
                    pltpu.async_copy(
                        table_hbm.at[idx_v.at[nbi]], rows_v.at[nbi], gsem[nbi]
                    )
                )
            gathers[i].wait()
            writes.append(
                pltpu.async_copy(
                    rows_v.at[bi],
                    out_hbm.at[pl.ds(base + i * CHUNK, CHUNK)],
                    wsem[bi],
                )
            )
        writes[n_chunks - 2].wait()
        writes[n_chunks - 1].wait()

    return body(flat_idx, table)


def kernel(x, table):
    b, t = x.shape
    flat_idx = x.reshape(-1).astype(jnp.int32)
    out = _gather_rows(flat_idx, table, b * t)
    return out.reshape(b, t, VOCAB_DIM)
